# baseline (speedup 1.0000x reference)
import functools

import jax
import jax.numpy as jnp
import numpy as np
from jax import lax
from jax.experimental import pallas as pl
from jax.experimental.pallas import tpu as pltpu
from jax.experimental.pallas import tpu_sc as plsc

B, C, N, H, W = 4, 4, 8, 512, 512
NPAIR = B * (C - 1)
HW = H * W
NC, NS = 2, 16
NW = NC * NS
HT_SPLIT = 256
CH = 4096
TC_HB = 256


def _pairs():
    return [(p // (C - 1), 1 + p % (C - 1)) for p in range(NPAIR)]


def _sc_body(lg_hbm, mk_hbm, out_hbm, mbuf, lbuf, pbuf, sem0, sem1):
    wid = lax.axis_index("s") * NC + lax.axis_index("c")
    S = (H - HT_SPLIT) * W // NW
    base = HT_SPLIT * W + wid * S

    chunks = []
    off = 0
    while off < S:
        sz = min(CH, S - off)
        chunks.append((off, sz))
        off += sz
    tasks = []
    for p, (b, c) in enumerate(_pairs()):
        for ci, (off, sz) in enumerate(chunks):
            tasks.append((p, b, c, off, sz, ci == len(chunks) - 1))

    sems = (sem0, sem1)

    def start(slot, task):
        p, b, c, off, sz, _ = task
        src0 = base + off
        copies = []
        for n in range(N):
            row = ((b * C + c) * N + n)
            cp = pltpu.make_async_copy(
                mk_hbm.at[pl.ds(row * HW + src0, sz)],
                mbuf.at[slot, n, pl.ds(0, sz)],
                sems[slot],
            )
            cp.start()
            copies.append(cp)
        lrow = b * C + c
        cp = pltpu.make_async_copy(
            lg_hbm.at[pl.ds(lrow * HW + src0, sz)],
            lbuf.at[slot, pl.ds(0, sz)],
            sems[slot],
        )
        cp.start()
        copies.append(cp)
        return copies

    def compute(slot, task, acc):
        _, _, _, _, sz, _ = task

        def body(i, acc):
            s = mbuf[slot, 0, pl.ds(i * 16, 16)]
            for n in range(1, N):
                s = s + mbuf[slot, n, pl.ds(i * 16, 16)]
            lgv = lbuf[slot, pl.ds(i * 16, 16)]
            return acc + jnp.where(s == 0.0, lgv, jnp.zeros_like(lgv))

        return lax.fori_loop(0, sz // 16, body, acc, unroll=2)

    pending = start(0, tasks[0])
    acc = jnp.zeros((16,), jnp.float32)
    for t, task in enumerate(tasks):
        nxt = None
        if t + 1 < len(tasks):
            nxt = start((t + 1) % 2, tasks[t + 1])
        for cp in pending:
            cp.wait()
        acc = compute(t % 2, task, acc)
        p, _, _, _, _, last = task
        if last:
            pbuf[p] = acc
            acc = jnp.zeros((16,), jnp.float32)
        pending = nxt

    pltpu.sync_copy(pbuf, out_hbm.at[wid])


def _sc_partials(lg_flat, mk_flat):
    mesh = plsc.VectorSubcoreMesh(core_axis_name="c", subcore_axis_name="s")
    ker = pl.kernel(
        _sc_body,
        out_type=jax.ShapeDtypeStruct((NW, NPAIR, 16), jnp.float32),
        mesh=mesh,
        scratch_types=[
            pltpu.VMEM((2, N, CH), jnp.float32),
            pltpu.VMEM((2, CH), jnp.float32),
            pltpu.VMEM((NPAIR, 16), jnp.float32),
            pltpu.SemaphoreType.DMA,
            pltpu.SemaphoreType.DMA,
        ],
    )
    return ker(lg_flat, mk_flat)


def _tc_body(logits_ref, masks_ref, out_ref):
    j = pl.program_id(1)

    @pl.when(j == 0)
    def _init():
        out_ref[...] = jnp.zeros_like(out_ref)

    lg = logits_ref[0, 0]
    masks = masks_ref[0, 0]
    covered = jnp.sum(masks, axis=0) > 0.0
    outside = jnp.where(covered, jnp.zeros_like(lg), lg)
    part = jnp.sum(outside.reshape(TC_HB, W // 128, 128), axis=(0, 1))
    out_ref[...] += part.reshape(1, 1, 128)


def _tc_partials(logits, box_masks):
    grid = (NPAIR, HT_SPLIT // TC_HB)
    return pl.pallas_call(
        _tc_body,
        grid=grid,
        in_specs=[
            pl.BlockSpec(
                (1, 1, TC_HB, W),
                lambda i, j: (i // (C - 1), 1 + i % (C - 1), j, 0),
            ),
            pl.BlockSpec(
                (1, 1, N, TC_HB, W),
                lambda i, j: (i // (C - 1), 1 + i % (C - 1), 0, j, 0),
            ),
        ],
        out_specs=pl.BlockSpec((1, 1, 128), lambda i, j: (i, 0, 0)),
        out_shape=jax.ShapeDtypeStruct((NPAIR, 1, 128), jnp.float32),
    )(logits, box_masks)


def _combine_body(sc_ref, tc_ref, ann_ref, out_ref):
    sc = sc_ref[...]
    outside = jnp.sum(sc, axis=(0, 2))
    outside = outside + jnp.sum(tc_ref[...], axis=(1, 2))
    err = jnp.where(outside >= 0.0, outside * outside, jnp.zeros_like(outside))
    err = err * ann_ref[0]
    out_ref[0, 0] = jnp.sum(err) / float(HW)


def _combine(sc_part, tc_part, annfg):
    return pl.pallas_call(
        _combine_body,
        in_specs=[
            pl.BlockSpec((NW, NPAIR, 16), lambda: (0, 0, 0)),
            pl.BlockSpec((NPAIR, 1, 128), lambda: (0, 0, 0)),
            pl.BlockSpec((1, NPAIR), lambda: (0, 0)),
        ],
        out_specs=pl.BlockSpec(memory_space=pltpu.SMEM),
        out_shape=jax.ShapeDtypeStruct((1, 1), jnp.float32),
    )(sc_part, tc_part, annfg)


def kernel(logits, box_masks, annotation_mask):
    lg_flat = logits.reshape(-1)
    mk_flat = box_masks.reshape(-1)
    sc_part = _sc_partials(lg_flat, mk_flat)
    tc_part = _tc_partials(logits, box_masks)
    annfg = annotation_mask[:, 1:].reshape(1, NPAIR)
    out = _combine(sc_part, tc_part, annfg)
    return out[0, 0]


# speedup vs baseline: 2.9273x; 2.9273x over previous
import functools

import jax
import jax.numpy as jnp
import numpy as np
from jax import lax
from jax.experimental import pallas as pl
from jax.experimental.pallas import tpu as pltpu
from jax.experimental.pallas import tpu_sc as plsc

B, C, N, H, W = 4, 4, 8, 512, 512
NPAIR = B * (C - 1)
HW = H * W
NC, NS = 2, 16
NW = NC * NS
HT_SPLIT = 256
CH = 4096
TC_HB = 256


def _pairs():
    return [(p // (C - 1), 1 + p % (C - 1)) for p in range(NPAIR)]


RPW = (H - HT_SPLIT) // NW


def _sc_body(lg_hbm, mk_hbm, out_hbm, mbuf, lbuf, pbuf, sem0, sem1):
    wid = lax.axis_index("s") * NC + lax.axis_index("c")
    row0 = HT_SPLIT + wid * RPW

    tasks = list(enumerate(_pairs()))
    sems = (sem0, sem1)

    def start(slot, task):
        _, (b, c) = task
        copies = []
        for n in range(N):
            cp = pltpu.make_async_copy(
                mk_hbm.at[b, c, n, pl.ds(row0, RPW), :],
                mbuf.at[slot, n],
                sems[slot],
            )
            cp.start()
            copies.append(cp)
        cp = pltpu.make_async_copy(
            lg_hbm.at[b, c, pl.ds(row0, RPW), :],
            lbuf.at[slot],
            sems[slot],
        )
        cp.start()
        copies.append(cp)
        return copies

    def compute(slot, acc):
        for r in range(RPW):
            def body(i, acc):
                s = mbuf[slot, 0, r, pl.ds(i * 16, 16)]
                for n in range(1, N):
                    s = s + mbuf[slot, n, r, pl.ds(i * 16, 16)]
                lgv = lbuf[slot, r, pl.ds(i * 16, 16)]
                return acc + jnp.where(s == 0.0, lgv, jnp.zeros_like(lgv))

            acc = lax.fori_loop(0, W // 16, body, acc, unroll=2)
        return acc

    pending = start(0, tasks[0])
    for t, task in enumerate(tasks):
        nxt = None
        if t + 1 < len(tasks):
            nxt = start((t + 1) % 2, tasks[t + 1])
        for cp in pending:
            cp.wait()
        acc = compute(t % 2, jnp.zeros((16,), jnp.float32))
        pbuf[task[0]] = acc
        pending = nxt

    pltpu.sync_copy(pbuf, out_hbm.at[wid])


def _sc_partials(logits, box_masks):
    mesh = plsc.VectorSubcoreMesh(core_axis_name="c", subcore_axis_name="s")
    ker = pl.kernel(
        _sc_body,
        out_type=jax.ShapeDtypeStruct((NW, NPAIR, 16), jnp.float32),
        mesh=mesh,
        scratch_types=[
            pltpu.VMEM((2, N, RPW, W), jnp.float32),
            pltpu.VMEM((2, RPW, W), jnp.float32),
            pltpu.VMEM((NPAIR, 16), jnp.float32),
            pltpu.SemaphoreType.DMA,
            pltpu.SemaphoreType.DMA,
        ],
    )
    return ker(logits, box_masks)


def _tc_body(logits_ref, masks_ref, out_ref):
    j = pl.program_id(1)

    @pl.when(j == 0)
    def _init():
        out_ref[...] = jnp.zeros_like(out_ref)

    lg = logits_ref[0, 0]
    masks = masks_ref[0, 0]
    covered = jnp.sum(masks, axis=0) > 0.0
    outside = jnp.where(covered, jnp.zeros_like(lg), lg)
    part = jnp.sum(outside.reshape(TC_HB, W // 128, 128), axis=(0, 1))
    out_ref[...] += part.reshape(1, 1, 128)


def _tc_partials(logits, box_masks):
    grid = (NPAIR, HT_SPLIT // TC_HB)
    return pl.pallas_call(
        _tc_body,
        grid=grid,
        in_specs=[
            pl.BlockSpec(
                (1, 1, TC_HB, W),
                lambda i, j: (i // (C - 1), 1 + i % (C - 1), j, 0),
            ),
            pl.BlockSpec(
                (1, 1, N, TC_HB, W),
                lambda i, j: (i // (C - 1), 1 + i % (C - 1), 0, j, 0),
            ),
        ],
        out_specs=pl.BlockSpec((1, 1, 128), lambda i, j: (i, 0, 0)),
        out_shape=jax.ShapeDtypeStruct((NPAIR, 1, 128), jnp.float32),
    )(logits, box_masks)


def _combine_body(sc_ref, tc_ref, ann_ref, out_ref):
    sc = sc_ref[...]
    outside = jnp.sum(sc, axis=(0, 2))
    outside = outside + jnp.sum(tc_ref[...], axis=(1, 2))
    err = jnp.where(outside >= 0.0, outside * outside, jnp.zeros_like(outside))
    err = err * ann_ref[0]
    out_ref[0, 0] = jnp.sum(err) / float(HW)


def _combine(sc_part, tc_part, annfg):
    return pl.pallas_call(
        _combine_body,
        in_specs=[
            pl.BlockSpec((NW, NPAIR, 16), lambda: (0, 0, 0)),
            pl.BlockSpec((NPAIR, 1, 128), lambda: (0, 0, 0)),
            pl.BlockSpec((1, NPAIR), lambda: (0, 0)),
        ],
        out_specs=pl.BlockSpec(memory_space=pltpu.SMEM),
        out_shape=jax.ShapeDtypeStruct((1, 1), jnp.float32),
    )(sc_part, tc_part, annfg)


def kernel(logits, box_masks, annotation_mask):
    sc_part = _sc_partials(logits, box_masks)
    tc_part = _tc_partials(logits, box_masks)
    annfg = annotation_mask[:, 1:].reshape(1, NPAIR)
    out = _combine(sc_part, tc_part, annfg)
    return out[0, 0]


# speedup vs baseline: 2.9984x; 1.0243x over previous
import jax
import jax.numpy as jnp
import numpy as np
from jax import lax
from jax.experimental import pallas as pl
from jax.experimental.pallas import tpu as pltpu
from jax.experimental.pallas import tpu_sc as plsc

B, C, N, H, W = 4, 4, 8, 512, 512
NPAIR = B * (C - 1)
NC, NS = 2, 16
NW = NC * NS
G = 24
assert (NPAIR * G) % NW == 0
K = NPAIR * G // NW
HT_SPLIT = H - 8 * G
TC_HB = HT_SPLIT


def _sc_body(lg_hbm, mk_hbm, out_hbm, mbuf, lbuf, pbuf, sem0, sem1):
    wid = lax.axis_index("s") * NC + lax.axis_index("c")
    sems = (sem0, sem1)

    def coords(k):
        t = wid * K + k
        p = t // G
        g = t % G
        b = p // (C - 1)
        c = 1 + p % (C - 1)
        return b, c, HT_SPLIT + g * 8

    def start(slot, k):
        b, c, row0 = coords(k)
        cps = [
            pltpu.make_async_copy(
                mk_hbm.at[b, c, :, pl.ds(row0, 8), :], mbuf.at[slot], sems[slot]
            ),
            pltpu.make_async_copy(
                lg_hbm.at[b, c, pl.ds(row0, 8), :], lbuf.at[slot], sems[slot]
            ),
        ]
        for cp in cps:
            cp.start()
        return cps

    def compute(slot):
        acc = jnp.zeros((16,), jnp.float32)
        for r in range(8):
            def body(i, acc):
                s = mbuf[slot, 0, r, pl.ds(i * 16, 16)]
                for n in range(1, N):
                    s = s + mbuf[slot, n, r, pl.ds(i * 16, 16)]
                lgv = lbuf[slot, r, pl.ds(i * 16, 16)]
                return acc + jnp.where(s == 0.0, lgv, jnp.zeros_like(lgv))

            acc = lax.fori_loop(0, W // 16, body, acc, unroll=4)
        return acc

    pending = start(0, 0)
    for k in range(K):
        nxt = start((k + 1) % 2, k + 1) if k + 1 < K else None
        for cp in pending:
            cp.wait()
        pbuf[k] = compute(k % 2)
        pending = nxt

    pltpu.sync_copy(pbuf, out_hbm.at[wid])


def _sc_partials(logits, box_masks):
    mesh = plsc.VectorSubcoreMesh(core_axis_name="c", subcore_axis_name="s")
    ker = pl.kernel(
        _sc_body,
        out_type=jax.ShapeDtypeStruct((NW, K, 16), jnp.float32),
        mesh=mesh,
        scratch_types=[
            pltpu.VMEM((2, N, 8, W), jnp.float32),
            pltpu.VMEM((2, 8, W), jnp.float32),
            pltpu.VMEM((K, 16), jnp.float32),
            pltpu.SemaphoreType.DMA,
            pltpu.SemaphoreType.DMA,
        ],
    )
    return ker(logits, box_masks)


def _tc_body(logits_ref, masks_ref, out_ref):
    j = pl.program_id(1)

    @pl.when(j == 0)
    def _init():
        out_ref[...] = jnp.zeros_like(out_ref)

    lg = logits_ref[0, 0]
    masks = masks_ref[0, 0]
    covered = jnp.sum(masks, axis=0) > 0.0
    outside = jnp.where(covered, jnp.zeros_like(lg), lg)
    part = jnp.sum(outside.reshape(TC_HB, W // 128, 128), axis=(0, 1))
    out_ref[...] += part.reshape(1, 1, 128)


def _tc_partials(logits, box_masks):
    grid = (NPAIR, HT_SPLIT // TC_HB)
    return pl.pallas_call(
        _tc_body,
        grid=grid,
        in_specs=[
            pl.BlockSpec(
                (1, 1, TC_HB, W),
                lambda i, j: (i // (C - 1), 1 + i % (C - 1), j, 0),
            ),
            pl.BlockSpec(
                (1, 1, N, TC_HB, W),
                lambda i, j: (i // (C - 1), 1 + i % (C - 1), 0, j, 0),
            ),
        ],
        out_specs=pl.BlockSpec((1, 1, 128), lambda i, j: (i, 0, 0)),
        out_shape=jax.ShapeDtypeStruct((NPAIR, 1, 128), jnp.float32),
    )(logits, box_masks)


def _task_onehot():
    oh = np.zeros((NW, K, NPAIR), np.float32)
    for w in range(NW):
        for k in range(K):
            oh[w, k, (w * K + k) // G] = 1.0
    return jnp.asarray(oh)


def _combine_body(sc_ref, tc_ref, ann_ref, oh_ref, out_ref):
    sc = jnp.sum(sc_ref[...], axis=2)
    outside = jnp.sum(sc[:, :, None] * oh_ref[...], axis=(0, 1))
    outside = outside + jnp.sum(tc_ref[...], axis=(1, 2))
    err = jnp.where(outside >= 0.0, outside * outside, jnp.zeros_like(outside))
    err = err * ann_ref[0]
    out_ref[0, 0] = jnp.sum(err) / float(H * W)


def _combine(sc_part, tc_part, annfg, onehot):
    return pl.pallas_call(
        _combine_body,
        in_specs=[
            pl.BlockSpec((NW, K, 16), lambda: (0, 0, 0)),
            pl.BlockSpec((NPAIR, 1, 128), lambda: (0, 0, 0)),
            pl.BlockSpec((1, NPAIR), lambda: (0, 0)),
            pl.BlockSpec((NW, K, NPAIR), lambda: (0, 0, 0)),
        ],
        out_specs=pl.BlockSpec(memory_space=pltpu.SMEM),
        out_shape=jax.ShapeDtypeStruct((1, 1), jnp.float32),
    )(sc_part, tc_part, annfg, onehot)


def kernel(logits, box_masks, annotation_mask):
    sc_part = _sc_partials(logits, box_masks)
    tc_part = _tc_partials(logits, box_masks)
    annfg = annotation_mask[:, 1:].reshape(1, NPAIR)
    out = _combine(sc_part, tc_part, annfg, _task_onehot())
    return out[0, 0]


# speedup vs baseline: 3.1830x; 1.0616x over previous
import jax
import jax.numpy as jnp
import numpy as np
from jax import lax
from jax.experimental import pallas as pl
from jax.experimental.pallas import tpu as pltpu
from jax.experimental.pallas import tpu_sc as plsc

B, C, N, H, W = 4, 4, 8, 512, 512
NPAIR = B * (C - 1)
NC, NS = 2, 16
NW = NC * NS
G = 16
assert (NPAIR * G) % NW == 0
K = NPAIR * G // NW
HT_SPLIT = H - 8 * G
TC_HB = HT_SPLIT


NBUF = 3


def _sc_body(lg_hbm, mk_hbm, out_hbm, mbuf, lbuf, pbuf, sem0, sem1, sem2):
    wid = lax.axis_index("s") * NC + lax.axis_index("c")
    sems = (sem0, sem1, sem2)

    def coords(k):
        t = wid * K + k
        p = t // G
        g = t % G
        b = p // (C - 1)
        c = 1 + p % (C - 1)
        return b, c, HT_SPLIT + g * 8

    def start(slot, k):
        b, c, row0 = coords(k)
        cps = [
            pltpu.make_async_copy(
                mk_hbm.at[b, c, :, pl.ds(row0, 8), :], mbuf.at[slot], sems[slot]
            ),
            pltpu.make_async_copy(
                lg_hbm.at[b, c, pl.ds(row0, 8), :], lbuf.at[slot], sems[slot]
            ),
        ]
        for cp in cps:
            cp.start()
        return cps

    def compute(slot):
        acc = jnp.zeros((16,), jnp.float32)
        for r in range(8):
            def body(i, acc):
                s = mbuf[slot, 0, r, pl.ds(i * 16, 16)]
                for n in range(1, N):
                    s = s + mbuf[slot, n, r, pl.ds(i * 16, 16)]
                lgv = lbuf[slot, r, pl.ds(i * 16, 16)]
                return acc + jnp.where(s == 0.0, lgv, jnp.zeros_like(lgv))

            acc = lax.fori_loop(0, W // 16, body, acc, unroll=4)
        return acc

    ring = [start(s, s) for s in range(min(NBUF, K))]
    for k in range(K):
        for cp in ring[k % NBUF]:
            cp.wait()
        pbuf[k] = compute(k % NBUF)
        if k + NBUF < K:
            ring[k % NBUF] = start(k % NBUF, k + NBUF)

    pltpu.sync_copy(pbuf, out_hbm.at[wid])


def _sc_partials(logits, box_masks):
    mesh = plsc.VectorSubcoreMesh(core_axis_name="c", subcore_axis_name="s")
    ker = pl.kernel(
        _sc_body,
        out_type=jax.ShapeDtypeStruct((NW, K, 16), jnp.float32),
        mesh=mesh,
        scratch_types=[
            pltpu.VMEM((NBUF, N, 8, W), jnp.float32),
            pltpu.VMEM((NBUF, 8, W), jnp.float32),
            pltpu.VMEM((K, 16), jnp.float32),
            pltpu.SemaphoreType.DMA,
            pltpu.SemaphoreType.DMA,
            pltpu.SemaphoreType.DMA,
        ],
    )
    return ker(logits, box_masks)


def _tc_body(logits_ref, masks_ref, out_ref):
    j = pl.program_id(1)

    @pl.when(j == 0)
    def _init():
        out_ref[...] = jnp.zeros_like(out_ref)

    lg = logits_ref[0, 0]
    masks = masks_ref[0, 0]
    covered = jnp.sum(masks, axis=0) > 0.0
    outside = jnp.where(covered, jnp.zeros_like(lg), lg)
    part = jnp.sum(outside.reshape(TC_HB, W // 128, 128), axis=(0, 1))
    out_ref[...] += part.reshape(1, 1, 128)


def _tc_partials(logits, box_masks):
    grid = (NPAIR, HT_SPLIT // TC_HB)
    return pl.pallas_call(
        _tc_body,
        grid=grid,
        in_specs=[
            pl.BlockSpec(
                (1, 1, TC_HB, W),
                lambda i, j: (i // (C - 1), 1 + i % (C - 1), j, 0),
            ),
            pl.BlockSpec(
                (1, 1, N, TC_HB, W),
                lambda i, j: (i // (C - 1), 1 + i % (C - 1), 0, j, 0),
            ),
        ],
        out_specs=pl.BlockSpec((1, 1, 128), lambda i, j: (i, 0, 0)),
        out_shape=jax.ShapeDtypeStruct((NPAIR, 1, 128), jnp.float32),
    )(logits, box_masks)


def _task_onehot():
    oh = np.zeros((NW, K, NPAIR), np.float32)
    for w in range(NW):
        for k in range(K):
            oh[w, k, (w * K + k) // G] = 1.0
    return jnp.asarray(oh)


def _combine_body(sc_ref, tc_ref, ann_ref, oh_ref, out_ref):
    sc = jnp.sum(sc_ref[...], axis=2)
    outside = jnp.sum(sc[:, :, None] * oh_ref[...], axis=(0, 1))
    outside = outside + jnp.sum(tc_ref[...], axis=(1, 2))
    err = jnp.where(outside >= 0.0, outside * outside, jnp.zeros_like(outside))
    err = err * ann_ref[0]
    out_ref[0, 0] = jnp.sum(err) / float(H * W)


def _combine(sc_part, tc_part, annfg, onehot):
    return pl.pallas_call(
        _combine_body,
        in_specs=[
            pl.BlockSpec((NW, K, 16), lambda: (0, 0, 0)),
            pl.BlockSpec((NPAIR, 1, 128), lambda: (0, 0, 0)),
            pl.BlockSpec((1, NPAIR), lambda: (0, 0)),
            pl.BlockSpec((NW, K, NPAIR), lambda: (0, 0, 0)),
        ],
        out_specs=pl.BlockSpec(memory_space=pltpu.SMEM),
        out_shape=jax.ShapeDtypeStruct((1, 1), jnp.float32),
    )(sc_part, tc_part, annfg, onehot)


def kernel(logits, box_masks, annotation_mask):
    sc_part = _sc_partials(logits, box_masks)
    tc_part = _tc_partials(logits, box_masks)
    annfg = annotation_mask[:, 1:].reshape(1, NPAIR)
    out = _combine(sc_part, tc_part, annfg, _task_onehot())
    return out[0, 0]


# speedup vs baseline: 4.7733x; 1.4996x over previous
import jax
import jax.numpy as jnp
import numpy as np
from jax.experimental import pallas as pl
from jax.experimental.pallas import tpu as pltpu

B, C, N, H, W = 4, 4, 8, 512, 512
Hb = 512


def _body(ann_ref, logits_ref, masks_ref, out_ref, acc_ref):
    i = pl.program_id(0)
    j = pl.program_id(1)
    n_j = pl.num_programs(1)

    @pl.when(jnp.logical_and(i == 0, j == 0))
    def _init_out():
        out_ref[0, 0] = 0.0

    @pl.when(j == 0)
    def _init_acc():
        acc_ref[0, 0] = 0.0

    lg = logits_ref[0, 0]
    masks = masks_ref[0, 0]
    covered = jnp.sum(masks, axis=0) > 0.0
    outside = jnp.where(covered, jnp.zeros_like(lg), lg)
    acc_ref[0, 0] += jnp.sum(outside)

    @pl.when(j == n_j - 1)
    def _finish_pair():
        o = acc_ref[0, 0]
        b = i // (C - 1)
        c = i % (C - 1)
        err = jnp.where(o >= 0.0, o * o, 0.0) * ann_ref[b, c + 1]
        out_ref[0, 0] += err


def kernel(logits, box_masks, annotation_mask):
    n_pairs = B * (C - 1)
    grid = (n_pairs, H // Hb)

    out = pl.pallas_call(
        _body,
        grid=grid,
        in_specs=[
            pl.BlockSpec(memory_space=pltpu.SMEM),
            pl.BlockSpec(
                (1, 1, Hb, W),
                lambda i, j: (i // (C - 1), 1 + i % (C - 1), j, 0),
            ),
            pl.BlockSpec(
                (1, 1, N, Hb, W),
                lambda i, j: (i // (C - 1), 1 + i % (C - 1), 0, j, 0),
            ),
        ],
        out_specs=pl.BlockSpec(memory_space=pltpu.SMEM),
        out_shape=jax.ShapeDtypeStruct((1, 1), jnp.float32),
        scratch_shapes=[pltpu.SMEM((1, 1), jnp.float32)],
    )(annotation_mask, logits, box_masks)

    im_size = float(np.prod(logits.shape[2:]))
    return out[0, 0] / im_size


# speedup vs baseline: 4.8204x; 1.0099x over previous
import jax
import jax.numpy as jnp
import numpy as np
from jax.experimental import pallas as pl
from jax.experimental.pallas import tpu as pltpu

B, C, N, H, W = 4, 4, 8, 512, 512
Hb = 512


def _body(ann_ref, logits_ref, masks_ref, out_ref, acc_ref):
    i = pl.program_id(0)
    j = pl.program_id(1)
    n_j = pl.num_programs(1)

    @pl.when(jnp.logical_and(i == 0, j == 0))
    def _init_out():
        out_ref[0, 0] = 0.0

    @pl.when(j == 0)
    def _init_acc():
        acc_ref[0, 0] = 0.0

    lg = logits_ref[0, 0]
    masks = masks_ref[0, 0]
    covered = jnp.sum(masks, axis=0) > 0.0
    outside = jnp.where(covered, jnp.zeros_like(lg), lg)
    ones = jnp.ones((1, Hb), jnp.float32)
    colsum = jax.lax.dot_general(
        ones, outside, (((1,), (0,)), ((), ())),
        preferred_element_type=jnp.float32,
    )
    acc_ref[0, 0] += jnp.sum(colsum)

    @pl.when(j == n_j - 1)
    def _finish_pair():
        o = acc_ref[0, 0]
        b = i // (C - 1)
        c = i % (C - 1)
        err = jnp.where(o >= 0.0, o * o, 0.0) * ann_ref[b, c + 1]
        out_ref[0, 0] += err


def kernel(logits, box_masks, annotation_mask):
    n_pairs = B * (C - 1)
    grid = (n_pairs, H // Hb)

    out = pl.pallas_call(
        _body,
        grid=grid,
        in_specs=[
            pl.BlockSpec(memory_space=pltpu.SMEM),
            pl.BlockSpec(
                (1, 1, Hb, W),
                lambda i, j: (i // (C - 1), 1 + i % (C - 1), j, 0),
            ),
            pl.BlockSpec(
                (1, 1, N, Hb, W),
                lambda i, j: (i // (C - 1), 1 + i % (C - 1), 0, j, 0),
            ),
        ],
        out_specs=pl.BlockSpec(memory_space=pltpu.SMEM),
        out_shape=jax.ShapeDtypeStruct((1, 1), jnp.float32),
        scratch_shapes=[pltpu.SMEM((1, 1), jnp.float32)],
    )(annotation_mask, logits, box_masks)

    im_size = float(np.prod(logits.shape[2:]))
    return out[0, 0] / im_size


# speedup vs baseline: 4.8386x; 1.0038x over previous
import jax
import jax.numpy as jnp
import numpy as np
from jax.experimental import pallas as pl
from jax.experimental.pallas import tpu as pltpu

B, C, N, H, W = 4, 4, 8, 512, 512
Hb = 512


def _body(ann_ref, logits_ref, masks_ref, out_ref, acc_ref):
    i = pl.program_id(0)
    j = pl.program_id(1)
    n_j = pl.num_programs(1)

    @pl.when(jnp.logical_and(i == 0, j == 0))
    def _init_out():
        out_ref[0, 0] = 0.0

    @pl.when(j == 0)
    def _init_acc():
        acc_ref[0, 0] = 0.0

    lg = logits_ref[0, 0]
    masks = masks_ref[0, 0]
    covered = jnp.sum(masks, axis=0) > 0.0
    outside = jnp.where(covered, jnp.zeros_like(lg), lg)
    acc_ref[0, 0] += jnp.sum(outside)

    @pl.when(j == n_j - 1)
    def _finish_pair():
        o = acc_ref[0, 0]
        b = i // (C - 1)
        c = i % (C - 1)
        err = jnp.where(o >= 0.0, o * o, 0.0) * ann_ref[b, c + 1]
        out_ref[0, 0] += err


def kernel(logits, box_masks, annotation_mask):
    n_pairs = B * (C - 1)
    grid = (n_pairs, H // Hb)

    out = pl.pallas_call(
        _body,
        grid=grid,
        in_specs=[
            pl.BlockSpec(memory_space=pltpu.SMEM),
            pl.BlockSpec(
                (1, 1, Hb, W),
                lambda i, j: (i // (C - 1), 1 + i % (C - 1), j, 0),
            ),
            pl.BlockSpec(
                (1, 1, N, Hb, W),
                lambda i, j: (i // (C - 1), 1 + i % (C - 1), 0, j, 0),
            ),
        ],
        out_specs=pl.BlockSpec(memory_space=pltpu.SMEM),
        out_shape=jax.ShapeDtypeStruct((1, 1), jnp.float32),
        scratch_shapes=[pltpu.SMEM((1, 1), jnp.float32)],
    )(annotation_mask, logits, box_masks)

    im_size = float(np.prod(logits.shape[2:]))
    return out[0, 0] / im_size
